# baseline (speedup 1.0000x reference)
import functools

import jax
import jax.numpy as jnp
from jax import lax
from jax.experimental import pallas as pl
from jax.experimental.pallas import tpu as pltpu
from jax.experimental.pallas import tpu_sc as plsc

_NC = 2
_NS = 16
_NW = _NC * _NS
_D = 16
_R = 2.0
_T = 1.0


def _sc_gather(table, idx3):
    nw, k, w = idx3.shape
    rows_per_w = k * w

    mesh = plsc.VectorSubcoreMesh(core_axis_name="c", subcore_axis_name="s")

    @functools.partial(
        pl.kernel,
        mesh=mesh,
        out_type=jax.ShapeDtypeStruct((nw * rows_per_w, _D), jnp.float32),
        compiler_params=pltpu.CompilerParams(use_tc_tiling_on_sc=False),
        scratch_types=[
            pltpu.VMEM((k, w), jnp.int32),
            pltpu.VMEM((rows_per_w, _D), jnp.float32),
            pltpu.SemaphoreType.DMA,
        ],
    )
    def gather_kernel(table_hbm, idx_hbm, out_hbm, idx_v, rows_v, sem):
        wid = lax.axis_index("s") * _NC + lax.axis_index("c")
        pltpu.sync_copy(idx_hbm.at[wid], idx_v)
        copies = []
        for j in range(k):
            copies.append(
                pltpu.async_copy(
                    table_hbm.at[idx_v.at[j]],
                    rows_v.at[pl.ds(j * w, w)],
                    sem,
                )
            )
        for c in copies:
            c.wait()
        pltpu.sync_copy(rows_v, out_hbm.at[pl.ds(wid * rows_per_w, rows_per_w)])

    return gather_kernel(table, idx3)


def _tc_tail_body(uv_ref, lab_ref, out_ref):
    uv = uv_ref[...]
    us = uv[:, :_D]
    vs = uv[:, _D:]
    d = us - vs
    e2 = jnp.sum(d * d, axis=1)
    un = jnp.sum(us * us, axis=1)
    vn = jnp.sum(vs * vs, axis=1)
    x = 1.0 + 2.0 * e2 / ((1.0 - un) * (1.0 - vn))
    dist = jnp.log(x + jnp.sqrt(x * x - 1.0))
    z = jnp.exp((dist - _R) / _T)
    lab = lab_ref[...]
    out_ref[...] = jnp.where(lab == 1, jnp.log(z + 1.0), jnp.log(1.0 + 1.0 / z))


def _tc_tail(uv, labels):
    b = labels.shape[0]
    return pl.pallas_call(
        _tc_tail_body,
        out_shape=jax.ShapeDtypeStruct((b,), jnp.float32),
    )(uv, labels)


@jax.jit
def kernel(pairs, labels, table):
    b = pairs.shape[0]
    idx = pairs.reshape(-1)
    k = (2 * b) // (_NW * 128)
    idx3 = idx.reshape(_NW, k, 128)
    uv = _sc_gather(table, idx3)
    uv2 = uv.reshape(b, 2 * _D)
    return _tc_tail(uv2, labels)


# speedup vs baseline: 3.5087x; 3.5087x over previous
import functools

import jax
import jax.numpy as jnp
from jax import lax
from jax.experimental import pallas as pl
from jax.experimental.pallas import tpu as pltpu
from jax.experimental.pallas import tpu_sc as plsc

_NC = 2
_NS = 16
_NW = _NC * _NS
_D = 16
_L = 16
_R = 2.0
_T = 1.0
_W = 16


def _sc_pair_x(tT, idx_flat):
    n_idx = idx_flat.shape[0]
    v = tT.shape[1]
    per_w = n_idx // _NW
    k = per_w // _W
    pairs_per_w = per_w // 2

    mesh = plsc.VectorSubcoreMesh(core_axis_name="c", subcore_axis_name="s")

    @functools.partial(
        pl.kernel,
        mesh=mesh,
        out_type=jax.ShapeDtypeStruct((n_idx // 2,), jnp.float32),
        compiler_params=pltpu.CompilerParams(needs_layout_passes=False),
        scratch_types=[
            pltpu.VMEM((per_w,), jnp.int32),
            pltpu.VMEM((_W, _D, 128), jnp.float32),
            pltpu.VMEM((_W, _D, 128), jnp.float32),
            pltpu.VMEM((2 * _W * _D,), jnp.float32),
            pltpu.VMEM((pairs_per_w,), jnp.float32),
            pltpu.SemaphoreType.DMA,
            pltpu.SemaphoreType.DMA,
        ],
    )
    def sc_kernel(tab_hbm, idx_hbm, out_hbm, idx_v,
                  buf0, buf1, rows_v, x_v, sem0, sem1):
        wid = lax.axis_index("s") * _NC + lax.axis_index("c")
        base = wid * per_w
        pltpu.sync_copy(idx_hbm.at[pl.ds(base, per_w)], idx_v)

        iota = lax.iota(jnp.int32, _L)
        u_base = iota * (2 * _D)
        v_base = u_base + _D
        zero = jnp.zeros((_L,), jnp.float32)

        def win_base(node):
            return pl.multiple_of((node >> 7) << 7, 128)

        def fire_chunk(jj, buf, sem):
            vec = idx_v[pl.ds(jj * _W, _W)]
            for s in range(_W):
                node = vec[s]
                pltpu.async_copy(
                    tab_hbm.at[:, pl.ds(win_base(node), 128)],
                    buf.at[s],
                    sem,
                )

        def drain_extract_chunk(jj, buf, sem, half):
            vec = idx_v[pl.ds(jj * _W, _W)]
            for s in range(_W):
                node = vec[s]
                pltpu.make_async_copy(
                    tab_hbm.at[:, pl.ds(win_base(node), 128)],
                    buf.at[s],
                    sem,
                ).wait()
            for s in range(_W):
                node = vec[s]
                lane = node & 127
                feats = plsc.load_gather(
                    buf,
                    [jnp.full((_L,), s, jnp.int32), iota,
                     jnp.full((_L,), lane, jnp.int32)],
                )
                rows_v[pl.ds((half * _W + s) * _D, _D)] = feats

        def compute_block(bb):
            e2 = zero
            un = zero
            vn = zero
            for d in range(_D):
                uc = plsc.load_gather(rows_v, [u_base + d])
                vc = plsc.load_gather(rows_v, [v_base + d])
                df = uc - vc
                e2 = e2 + df * df
                un = un + uc * uc
                vn = vn + vc * vc
            xblk = 1.0 + 2.0 * e2 / ((1.0 - un) * (1.0 - vn))
            x_v[pl.ds(bb * _L, _L)] = xblk

        fire_chunk(0, buf0, sem0)
        fire_chunk(1, buf1, sem1)

        @pl.loop(0, k, step=2)
        def _(j):
            drain_extract_chunk(j, buf0, sem0, 0)

            @pl.when(j + 2 < k)
            def _():
                fire_chunk(j + 2, buf0, sem0)

            drain_extract_chunk(j + 1, buf1, sem1, 1)

            @pl.when(j + 3 < k)
            def _():
                fire_chunk(j + 3, buf1, sem1)

            compute_block(j // 2)

        pltpu.sync_copy(x_v, out_hbm.at[pl.ds(wid * pairs_per_w, pairs_per_w)])

    return sc_kernel(tT, idx_flat)


def _tc_tail_body(x_ref, lab_ref, out_ref):
    x = x_ref[...]
    dist = jnp.log(x + jnp.sqrt(x * x - 1.0))
    z = jnp.exp((dist - _R) / _T)
    lab = lab_ref[...]
    out_ref[...] = jnp.where(lab == 1, jnp.log(z + 1.0), jnp.log(1.0 + 1.0 / z))


def _tc_tail(x, labels):
    b = labels.shape[0]
    r = b // 128
    out = pl.pallas_call(
        _tc_tail_body,
        out_shape=jax.ShapeDtypeStruct((r, 128), jnp.float32),
    )(x.reshape(r, 128), labels.reshape(r, 128))
    return out.reshape(b)


@jax.jit
def kernel(pairs, labels, table):
    tT = table.T
    idx = pairs.reshape(-1)
    x = _sc_pair_x(tT, idx)
    return _tc_tail(x, labels)
